# baseline (speedup 1.0000x reference)
from functools import partial

import jax
import jax.numpy as jnp
from jax.experimental import pallas as pl
from jax.experimental.pallas import tpu as pltpu


def _round_up(x, m):
    return ((x + m - 1) // m) * m


_NBUF = 4
_NSPLIT = 4


def _fused_bilstm_kernel(
    x_any,
    xtr_ref,
    wihf_ref,
    bf_ref,
    wihb_ref,
    bb_ref,
    whhf_ref,
    wt_ref,
    btb_ref,
    w1_ref,
    b1_ref,
    w2_ref,
    b2_ref,
    out_ref,
    xbuf,
    wihs_ref,
    whhs_ref,
    sem,
    *,
    T: int,
    H: int,
    Bt: int,
):
    Bq = Bt // _NSPLIT

    def start_fetch(t):
        s = t % _NBUF
        for j in range(_NSPLIT):
            pltpu.make_async_copy(
                x_any.at[pl.ds(j * Bq, Bq), t, :],
                xbuf.at[s, pl.ds(j * Bq, Bq), :],
                sem.at[s, j],
            ).start()

    def wait_fetch(t):
        s = t % _NBUF
        for j in range(_NSPLIT):
            pltpu.make_async_copy(
                x_any.at[pl.ds(0, Bq), 0, :],
                xbuf.at[s, pl.ds(j * Bq, Bq), :],
                sem.at[s, j],
            ).wait()

    for t in range(min(_NBUF, T)):
        start_fetch(t)

    lane = jax.lax.broadcasted_iota(jnp.int32, (1, 4 * H), 1)
    half_mask = jnp.where((lane >= 2 * H) & (lane < 3 * H), 1.0, 0.5)
    wihs_ref[...] = (wihf_ref[...] * half_mask).astype(jnp.bfloat16)
    whhs_ref[...] = whhf_ref[...] * half_mask
    b = bf_ref[...] * half_mask

    whh = whhs_ref[...]

    h = jnp.zeros((Bt, H), jnp.float32)
    c = jnp.zeros((Bt, H), jnp.float32)
    x_t = None
    for t in range(T):
        wait_fetch(t)
        x_t = xbuf[t % _NBUF].astype(jnp.bfloat16)
        if t + _NBUF < T:
            start_fetch(t + _NBUF)
        gates = (
            jnp.dot(x_t, wihs_ref[...], preferred_element_type=jnp.float32)
            + jnp.dot(h, whh, preferred_element_type=jnp.float32)
            + b
        )
        ti = jnp.tanh(gates[:, 0:H])
        tf = jnp.tanh(gates[:, H:2 * H])
        g = jnp.tanh(gates[:, 2 * H:3 * H])
        to = jnp.tanh(gates[:, 3 * H:4 * H])
        c = 0.5 * ((1.0 + tf) * c + (1.0 + ti) * g)
        h = (0.5 * (1.0 + to)) * jnp.tanh(c)

    gb = (
        jnp.dot(x_t, wihb_ref[...].astype(jnp.bfloat16), preferred_element_type=jnp.float32)
        + bb_ref[...]
    )
    ti_b = jnp.tanh(gb[:, 0:H] * 0.5)
    g_b = jnp.tanh(gb[:, 2 * H:3 * H])
    to_b = jnp.tanh(gb[:, 3 * H:4 * H] * 0.5)
    c_b = (0.5 * (1.0 + ti_b)) * g_b
    h_b = (0.5 * (1.0 + to_b)) * jnp.tanh(c_b)

    track = jnp.maximum(
        jnp.dot(xtr_ref[...], wt_ref[...], preferred_element_type=jnp.float32)
        + btb_ref[...],
        0.0,
    )

    pre = (
        jnp.dot(h, w1_ref[0:H, :], preferred_element_type=jnp.float32)
        + jnp.dot(h_b, w1_ref[H:2 * H, :], preferred_element_type=jnp.float32)
        + jnp.dot(track, w1_ref[2 * H:3 * H, :], preferred_element_type=jnp.float32)
        + b1_ref[...]
    )
    hidden = jnp.maximum(pre, 0.0)
    out = (
        jnp.dot(hidden, w2_ref[...], preferred_element_type=jnp.float32)
        + b2_ref[...]
    )
    out_ref[...] = out.astype(out_ref.dtype)


@jax.jit
def kernel(x_seq, x_track, wih_f, b_f, wih_b, b_b, whh_f, wt, bt, w1, b1, w2p, b2p):
    B, T, Din = x_seq.shape
    Dtrk = x_track.shape[1]
    H = whh_f.shape[0]

    B_pad = _round_up(B, 8)
    if B_pad != B:
        x_seq = jnp.pad(x_seq, ((0, B_pad - B), (0, 0), (0, 0)))
        x_track = jnp.pad(x_track, ((0, B_pad - B), (0, 0)))

    out = pl.pallas_call(
        partial(_fused_bilstm_kernel, T=T, H=H, Bt=B_pad),
        out_shape=jax.ShapeDtypeStruct((B_pad, 128), jnp.float32),
        grid=(1,),
        in_specs=[
            pl.BlockSpec(memory_space=pltpu.MemorySpace.HBM),
            pl.BlockSpec((B_pad, Dtrk), lambda i: (0, 0)),
            pl.BlockSpec((Din, 4 * H), lambda i: (0, 0)),
            pl.BlockSpec((1, 4 * H), lambda i: (0, 0)),
            pl.BlockSpec((Din, 4 * H), lambda i: (0, 0)),
            pl.BlockSpec((1, 4 * H), lambda i: (0, 0)),
            pl.BlockSpec((H, 4 * H), lambda i: (0, 0)),
            pl.BlockSpec((Dtrk, H), lambda i: (0, 0)),
            pl.BlockSpec((1, H), lambda i: (0, 0)),
            pl.BlockSpec((3 * H, 64), lambda i: (0, 0)),
            pl.BlockSpec((1, 64), lambda i: (0, 0)),
            pl.BlockSpec((64, 128), lambda i: (0, 0)),
            pl.BlockSpec((1, 128), lambda i: (0, 0)),
        ],
        out_specs=pl.BlockSpec((B_pad, 128), lambda i: (0, 0)),
        scratch_shapes=[
            pltpu.VMEM((_NBUF, B_pad, Din), jnp.float32),
            pltpu.VMEM((Din, 4 * H), jnp.bfloat16),
            pltpu.VMEM((H, 4 * H), jnp.float32),
            pltpu.SemaphoreType.DMA((_NBUF, _NSPLIT)),
        ],
        compiler_params=pltpu.CompilerParams(
            dimension_semantics=("arbitrary",),
            vmem_limit_bytes=64 * 1024 * 1024,
        ),
    )(x_seq, x_track, wih_f, b_f, wih_b, b_b, whh_f, wt, bt, w1, b1, w2p, b2p)

    return out[:B, :3]


# speedup vs baseline: 1.0933x; 1.0933x over previous
from functools import partial

import jax
import jax.numpy as jnp
from jax.experimental import pallas as pl
from jax.experimental.pallas import tpu as pltpu


def _round_up(x, m):
    return ((x + m - 1) // m) * m


_NBUF = 4
_NSPLIT = 4


def _fused_bilstm_kernel(
    x_any,
    xtr_ref,
    wihf_ref,
    bf_ref,
    wihb_ref,
    bb_ref,
    whhf_ref,
    wt_ref,
    btb_ref,
    w1_ref,
    b1_ref,
    w2_ref,
    b2_ref,
    out_ref,
    xbuf,
    wihs_ref,
    whhs_ref,
    sem,
    *,
    T: int,
    H: int,
    Bt: int,
):
    Bq = Bt // _NSPLIT

    def start_fetch(t):
        s = t % _NBUF
        for j in range(_NSPLIT):
            pltpu.make_async_copy(
                x_any.at[pl.ds(j * Bq, Bq), t, :],
                xbuf.at[s, pl.ds(j * Bq, Bq), :],
                sem.at[s, j],
            ).start()

    def wait_fetch(t):
        s = t % _NBUF
        for j in range(_NSPLIT):
            pltpu.make_async_copy(
                x_any.at[pl.ds(0, Bq), 0, :],
                xbuf.at[s, pl.ds(j * Bq, Bq), :],
                sem.at[s, j],
            ).wait()

    for t in range(min(_NBUF, T)):
        start_fetch(t)

    lane = jax.lax.broadcasted_iota(jnp.int32, (1, 4 * H), 1)
    half_mask = jnp.where((lane >= 2 * H) & (lane < 3 * H), 1.0, 0.5)
    wihs_ref[...] = wihf_ref[...] * half_mask
    whhs_ref[...] = whhf_ref[...] * half_mask
    b = bf_ref[...] * half_mask

    whh = whhs_ref[...]

    Bh = Bt // 2

    def lstm_step(x_half, h, c):
        gates = (
            jnp.dot(x_half, wihs_ref[...], preferred_element_type=jnp.float32)
            + jnp.dot(h, whh, preferred_element_type=jnp.float32)
            + b
        )
        ti = jnp.tanh(gates[:, 0:H])
        tf = jnp.tanh(gates[:, H:2 * H])
        g = jnp.tanh(gates[:, 2 * H:3 * H])
        to = jnp.tanh(gates[:, 3 * H:4 * H])
        c = 0.5 * ((1.0 + tf) * c + (1.0 + ti) * g)
        h = (0.5 * (1.0 + to)) * jnp.tanh(c)
        return h, c

    h0 = jnp.zeros((Bh, H), jnp.float32)
    c0 = jnp.zeros((Bh, H), jnp.float32)
    h1 = jnp.zeros((Bh, H), jnp.float32)
    c1 = jnp.zeros((Bh, H), jnp.float32)
    x_t = None
    for t in range(T):
        wait_fetch(t)
        x_t = xbuf[t % _NBUF]
        if t + _NBUF < T:
            start_fetch(t + _NBUF)
        h0, c0 = lstm_step(x_t[0:Bh], h0, c0)
        h1, c1 = lstm_step(x_t[Bh:Bt], h1, c1)
    h = jnp.concatenate([h0, h1], axis=0)

    gb = (
        jnp.dot(x_t, wihb_ref[...], preferred_element_type=jnp.float32)
        + bb_ref[...]
    )
    ti_b = jnp.tanh(gb[:, 0:H] * 0.5)
    g_b = jnp.tanh(gb[:, 2 * H:3 * H])
    to_b = jnp.tanh(gb[:, 3 * H:4 * H] * 0.5)
    c_b = (0.5 * (1.0 + ti_b)) * g_b
    h_b = (0.5 * (1.0 + to_b)) * jnp.tanh(c_b)

    track = jnp.maximum(
        jnp.dot(xtr_ref[...], wt_ref[...], preferred_element_type=jnp.float32)
        + btb_ref[...],
        0.0,
    )

    pre = (
        jnp.dot(h, w1_ref[0:H, :], preferred_element_type=jnp.float32)
        + jnp.dot(h_b, w1_ref[H:2 * H, :], preferred_element_type=jnp.float32)
        + jnp.dot(track, w1_ref[2 * H:3 * H, :], preferred_element_type=jnp.float32)
        + b1_ref[...]
    )
    hidden = jnp.maximum(pre, 0.0)
    out = (
        jnp.dot(hidden, w2_ref[...], preferred_element_type=jnp.float32)
        + b2_ref[...]
    )
    out_ref[...] = out.astype(out_ref.dtype)


@jax.jit
def kernel(x_seq, x_track, wih_f, b_f, wih_b, b_b, whh_f, wt, bt, w1, b1, w2p, b2p):
    B, T, Din = x_seq.shape
    Dtrk = x_track.shape[1]
    H = whh_f.shape[0]

    B_pad = _round_up(B, 8)
    if B_pad != B:
        x_seq = jnp.pad(x_seq, ((0, B_pad - B), (0, 0), (0, 0)))
        x_track = jnp.pad(x_track, ((0, B_pad - B), (0, 0)))

    out = pl.pallas_call(
        partial(_fused_bilstm_kernel, T=T, H=H, Bt=B_pad),
        out_shape=jax.ShapeDtypeStruct((B_pad, 128), jnp.float32),
        grid=(1,),
        in_specs=[
            pl.BlockSpec(memory_space=pltpu.MemorySpace.HBM),
            pl.BlockSpec((B_pad, Dtrk), lambda i: (0, 0)),
            pl.BlockSpec((Din, 4 * H), lambda i: (0, 0)),
            pl.BlockSpec((1, 4 * H), lambda i: (0, 0)),
            pl.BlockSpec((Din, 4 * H), lambda i: (0, 0)),
            pl.BlockSpec((1, 4 * H), lambda i: (0, 0)),
            pl.BlockSpec((H, 4 * H), lambda i: (0, 0)),
            pl.BlockSpec((Dtrk, H), lambda i: (0, 0)),
            pl.BlockSpec((1, H), lambda i: (0, 0)),
            pl.BlockSpec((3 * H, 64), lambda i: (0, 0)),
            pl.BlockSpec((1, 64), lambda i: (0, 0)),
            pl.BlockSpec((64, 128), lambda i: (0, 0)),
            pl.BlockSpec((1, 128), lambda i: (0, 0)),
        ],
        out_specs=pl.BlockSpec((B_pad, 128), lambda i: (0, 0)),
        scratch_shapes=[
            pltpu.VMEM((_NBUF, B_pad, Din), jnp.float32),
            pltpu.VMEM((Din, 4 * H), jnp.float32),
            pltpu.VMEM((H, 4 * H), jnp.float32),
            pltpu.SemaphoreType.DMA((_NBUF, _NSPLIT)),
        ],
        compiler_params=pltpu.CompilerParams(
            dimension_semantics=("arbitrary",),
            vmem_limit_bytes=64 * 1024 * 1024,
        ),
    )(x_seq, x_track, wih_f, b_f, wih_b, b_b, whh_f, wt, bt, w1, b1, w2p, b2p)

    return out[:B, :3]
